# baseline (speedup 1.0000x reference)
import jax
import jax.numpy as jnp
from jax import lax
from jax.experimental import pallas as pl
from jax.experimental.pallas import tpu as pltpu, tpu_sc as plsc

D = 64
E = 1048576
L = 16
NC, NS = 2, 16
NW = NC * NS
PER_W = E // NW
C = 128
G = C // L
NBUF = 4
P = NBUF - 1
IDXB = 2048
CPB = IDXB // C
NBLK = PER_W // IDXB


def _body(xu_hbm, xi_hbm, eli_hbm, out_hbm,
          idx_u, idx_i, urows, irows, outv, tsc,
          su0, su1, su2, su3, si0, si1, si2, si3):
    sem_u = [su0, su1, su2, su3]
    sem_i = [si0, si1, si2, si3]
    wid = lax.axis_index("s") * NC + lax.axis_index("c")
    wbase = wid * PER_W
    wrow = wid * (PER_W // C)
    lane = lax.broadcasted_iota(jnp.int32, (L,), 0)

    def block_body(k, carry):
        brow = wrow + k * CPB
        pltpu.sync_copy(eli_hbm.at[pl.ds(brow, CPB), :], idx_u)
        pltpu.sync_copy(eli_hbm.at[pl.ds(E // C + brow, CPB), :], idx_i)

        def start(j, s):
            pltpu.async_copy(xu_hbm.at[idx_u.at[j]], urows.at[s], sem_u[s])
            pltpu.async_copy(xi_hbm.at[idx_i.at[j]], irows.at[s], sem_i[s])

        def wait_rows(s):
            pltpu.make_async_copy(xu_hbm.at[pl.ds(0, C), :], urows.at[s],
                                  sem_u[s]).wait()
            pltpu.make_async_copy(xi_hbm.at[pl.ds(0, C), :], irows.at[s],
                                  sem_i[s]).wait()

        for p in range(P):
            start(p, p)

        def quad_body(q, c2):
            for b in range(NBUF):
                j = q * NBUF + b

                @pl.when(j + P < CPB)
                def _prefetch():
                    start(j + P, (b + P) % NBUF)

                wait_rows(b)
                obase = (k * CPB + j) * C

                def group_body(g, c3):
                    ebase = g * L
                    res = jnp.zeros((L,), jnp.float32)
                    for jj in range(L):
                        e = ebase + jj
                        m0 = urows[b, e, pl.ds(0, L)] * irows[b, e, pl.ds(0, L)]
                        m1 = urows[b, e, pl.ds(L, L)] * irows[b, e, pl.ds(L, L)]
                        m2 = urows[b, e, pl.ds(2 * L, L)] * irows[b, e, pl.ds(2 * L, L)]
                        m3 = urows[b, e, pl.ds(3 * L, L)] * irows[b, e, pl.ds(3 * L, L)]
                        acc = (m0 + m1) + (m2 + m3)
                        res = jnp.where(lane == jj, jnp.sum(acc), res)
                    outv[pl.ds(obase + ebase, L)] = res
                    return c3

                lax.fori_loop(0, G, group_body, 0)
            return c2

        lax.fori_loop(0, CPB // NBUF, quad_body, 0)
        return carry

    lax.fori_loop(0, NBLK, block_body, 0)
    pltpu.sync_copy(outv, out_hbm.at[pl.ds(wbase, PER_W)])


def kernel(x_user, x_item, edge_label_index):
    eli = edge_label_index.reshape(2 * (E // C), C)
    mesh = plsc.VectorSubcoreMesh(core_axis_name="c", subcore_axis_name="s",
                                  num_cores=NC, num_subcores=NS)
    f = pl.kernel(
        _body,
        out_type=jax.ShapeDtypeStruct((E,), jnp.float32),
        mesh=mesh,
        scratch_types=[
            pltpu.VMEM((CPB, C), jnp.int32),
            pltpu.VMEM((CPB, C), jnp.int32),
            pltpu.VMEM((NBUF, C, D), jnp.float32),
            pltpu.VMEM((NBUF, C, D), jnp.float32),
            pltpu.VMEM((PER_W,), jnp.float32),
            pltpu.VMEM((G * L * L,), jnp.float32),
        ] + [pltpu.SemaphoreType.DMA] * (2 * NBUF),
        compiler_params=pltpu.CompilerParams(needs_layout_passes=False,
                                             use_tc_tiling_on_sc=False),
    )
    return f(x_user, x_item, eli)


# speedup vs baseline: 1.2930x; 1.2930x over previous
import jax
import jax.numpy as jnp
from jax import lax
from jax.experimental import pallas as pl
from jax.experimental.pallas import tpu as pltpu, tpu_sc as plsc

D = 64
E = 1048576
L = 16
NC, NS = 2, 16
NW = NC * NS
PER_W = E // NW
C = 128
G = C // L
NBUF = 4
P = NBUF - 1
IDXB = 2048
CPB = IDXB // C
NBLK = PER_W // IDXB
NROW = E // C


def _body(xu_hbm, xi_hbm, eli_hbm, out_hbm,
          idx_u, idx_i, urows, irows, outv, tsc,
          su0, su1, su2, su3, si0, si1, si2, si3, sbu, sbi):
    sem_u = [su0, su1, su2, su3]
    sem_i = [si0, si1, si2, si3]
    wid = lax.axis_index("s") * NC + lax.axis_index("c")
    wbase = wid * PER_W
    wrow = wid * (PER_W // C)
    lane = lax.broadcasted_iota(jnp.int32, (L,), 0)

    def start(kb, j, s):
        pltpu.async_copy(xu_hbm.at[idx_u.at[kb, j]], urows.at[s], sem_u[s])
        pltpu.async_copy(xi_hbm.at[idx_i.at[kb, j]], irows.at[s], sem_i[s])

    def wait_rows(s):
        pltpu.make_async_copy(xu_hbm.at[pl.ds(0, C), :], urows.at[s],
                              sem_u[s]).wait()
        pltpu.make_async_copy(xi_hbm.at[pl.ds(0, C), :], irows.at[s],
                              sem_i[s]).wait()

    def wait_idx():
        pltpu.make_async_copy(eli_hbm.at[pl.ds(0, CPB), :], idx_u.at[0],
                              sbu).wait()
        pltpu.make_async_copy(eli_hbm.at[pl.ds(0, CPB), :], idx_i.at[0],
                              sbi).wait()

    pltpu.sync_copy(eli_hbm.at[pl.ds(wrow, CPB), :], idx_u.at[0])
    pltpu.sync_copy(eli_hbm.at[pl.ds(NROW + wrow, CPB), :], idx_i.at[0])
    for p in range(P):
        start(0, p, p)

    def block_body(k, carry):
        kb = k & 1
        brow = wrow + k * CPB

        @pl.when(k + 1 < NBLK)
        def _idx_prefetch():
            nrow = brow + CPB
            pltpu.async_copy(eli_hbm.at[pl.ds(nrow, CPB), :],
                             idx_u.at[kb ^ 1], sbu)
            pltpu.async_copy(eli_hbm.at[pl.ds(NROW + nrow, CPB), :],
                             idx_i.at[kb ^ 1], sbi)

        for j in range(CPB):
            s = j % NBUF
            if j + P < CPB:
                start(kb, j + P, (j + P) % NBUF)
            else:
                @pl.when(k + 1 < NBLK)
                def _cross_prefetch():
                    if j + P == CPB:
                        wait_idx()
                    start(kb ^ 1, j + P - CPB, (j + P) % NBUF)

            wait_rows(s)
            obase = (k * CPB + j) * C

            @plsc.parallel_loop(0, G, unroll=2)
            def group_body(g):
                ebase = g * L
                tbase = g * (L * L)
                for jj in range(L):
                    e = ebase + jj
                    m0 = urows[s, e, pl.ds(0, L)] * irows[s, e, pl.ds(0, L)]
                    m1 = urows[s, e, pl.ds(L, L)] * irows[s, e, pl.ds(L, L)]
                    m2 = urows[s, e, pl.ds(2 * L, L)] * irows[s, e, pl.ds(2 * L, L)]
                    m3 = urows[s, e, pl.ds(3 * L, L)] * irows[s, e, pl.ds(3 * L, L)]
                    acc = (m0 + m1) + (m2 + m3)
                    plsc.store_scatter(tsc, [tbase + lane * L + jj], acc)
                t = [tsc[pl.ds(tbase + l * L, L)] for l in range(L)]
                while len(t) > 1:
                    t = [t[2 * i] + t[2 * i + 1] for i in range(len(t) // 2)]
                outv[pl.ds(obase + ebase, L)] = t[0]

        return carry

    lax.fori_loop(0, NBLK, block_body, 0)
    pltpu.sync_copy(outv, out_hbm.at[pl.ds(wbase, PER_W)])


def kernel(x_user, x_item, edge_label_index):
    eli = edge_label_index.reshape(2 * NROW, C)
    mesh = plsc.VectorSubcoreMesh(core_axis_name="c", subcore_axis_name="s",
                                  num_cores=NC, num_subcores=NS)
    f = pl.kernel(
        _body,
        out_type=jax.ShapeDtypeStruct((E,), jnp.float32),
        mesh=mesh,
        scratch_types=[
            pltpu.VMEM((2, CPB, C), jnp.int32),
            pltpu.VMEM((2, CPB, C), jnp.int32),
            pltpu.VMEM((NBUF, C, D), jnp.float32),
            pltpu.VMEM((NBUF, C, D), jnp.float32),
            pltpu.VMEM((PER_W,), jnp.float32),
            pltpu.VMEM((G * L * L,), jnp.float32),
        ] + [pltpu.SemaphoreType.DMA] * (2 * NBUF + 2),
        compiler_params=pltpu.CompilerParams(needs_layout_passes=False,
                                             use_tc_tiling_on_sc=False),
    )
    return f(x_user, x_item, eli)
